# baseline (speedup 1.0000x reference)
import functools

import jax
import jax.numpy as jnp
from jax import lax
from jax.experimental import pallas as pl
from jax.experimental.pallas import tpu as pltpu
from jax.experimental.pallas import tpu_sc as plsc

NC = 2
NS = 16
L = 16
EPS = 1e-5


def _sc_degree(col, n):
    e = col.shape[0]
    nw = NC * NS
    ec = e // nw
    mesh = plsc.VectorSubcoreMesh(
        core_axis_name="c", subcore_axis_name="s", num_cores=NC, num_subcores=NS
    )

    @functools.partial(
        pl.kernel,
        mesh=mesh,
        out_type=jax.ShapeDtypeStruct((nw * n,), jnp.float32),
        scratch_types=[
            pltpu.VMEM((ec,), jnp.int32),
            pltpu.VMEM((n,), jnp.float32),
        ],
        compiler_params=pltpu.CompilerParams(needs_layout_passes=False),
    )
    def k(col_hbm, deg_hbm, colv, degv):
        wid = lax.axis_index("s") * NC + lax.axis_index("c")
        pltpu.sync_copy(col_hbm.at[pl.ds(wid * ec, ec)], colv)

        def zbody(i, carry):
            degv[pl.ds(i * L, L)] = jnp.zeros((L,), jnp.float32)
            return carry

        lax.fori_loop(0, n // L, zbody, 0, unroll=4)

        ones = jnp.ones((L,), jnp.float32)

        def cbody(i, carry):
            idx = colv[pl.ds(i * L, L)]
            plsc.addupdate_scatter(degv, [idx], ones)
            return carry

        lax.fori_loop(0, ec // L, cbody, 0, unroll=4)
        pltpu.sync_copy(degv, deg_hbm.at[pl.ds(wid * n, n)])

    return k(col).reshape(nw, n)


def _sc_aggregate(u, row, col, n):
    e = row.shape[0]
    c_dim = u.shape[1]
    ec = e // (NC * NS)
    K = 80
    nchunk = ec // K
    NBUF = 4
    ngrp = nchunk // NBUF
    assert nchunk % NBUF == 1 and (ngrp - 1) % 2 == 0
    npair = (ngrp - 1) // 2
    rpt = 632
    last = n - rpt * (NS - 1)
    mesh = plsc.VectorSubcoreMesh(
        core_axis_name="c", subcore_axis_name="s", num_cores=NC, num_subcores=NS
    )

    nci = nchunk + 3
    idx3 = jnp.zeros((NC * NS, nci, 2, K), jnp.int32)
    idx3 = idx3.at[:, :nchunk].set(
        jnp.stack(
            [row.reshape(NC * NS, nchunk, K), col.reshape(NC * NS, nchunk, K)],
            axis=2,
        )
    )

    @functools.partial(
        pl.kernel,
        mesh=mesh,
        out_type=jax.ShapeDtypeStruct((NC, n, c_dim), jnp.float32),
        scratch_types=[
            pltpu.VMEM((2, NBUF, 2, K), jnp.int32),
            pltpu.VMEM((NBUF, K, c_dim), jnp.float32),
            pltpu.VMEM_SHARED((n, c_dim), jnp.float32),
            pltpu.SemaphoreType.DMA,
            pltpu.SemaphoreType.DMA,
            pltpu.SemaphoreType.DMA,
        ],
        compiler_params=pltpu.CompilerParams(needs_layout_passes=False),
    )
    def k(u_hbm, idx_hbm, out_hbm, idxv, gbuf, acc, gsem, ssem, isem):
        cid = lax.axis_index("c")
        sid = lax.axis_index("s")
        wid = cid * NS + sid

        pltpu.async_copy(idx_hbm.at[wid, pl.ds(0, NBUF)], idxv.at[0], isem)
        pltpu.async_copy(idx_hbm.at[wid, pl.ds(NBUF, NBUF)], idxv.at[1], isem)

        def wait_idx(p):
            pltpu.make_async_copy(
                idx_hbm.at[wid, pl.ds(0, NBUF)], idxv.at[p], isem
            ).wait()

        nvec = c_dim // L

        def zbody(i, carry):
            r = i // nvec
            col0 = (i % nvec) * L
            gbuf[0, r, pl.ds(col0, L)] = jnp.zeros((L,), jnp.float32)
            return carry

        lax.fori_loop(0, K * nvec, zbody, 0, unroll=4)
        base_r = sid * rpt

        @pl.when(sid < NS - 1)
        def _():
            for m in range(rpt // K):
                pltpu.sync_copy(gbuf.at[0], acc.at[pl.ds(base_r + m * K, K)])
            rem = rpt % K
            pltpu.sync_copy(
                gbuf.at[0, pl.ds(0, rem)],
                acc.at[pl.ds(base_r + (rpt // K) * K, rem)],
            )

        @pl.when(sid == NS - 1)
        def _():
            for m in range(last // K):
                pltpu.sync_copy(gbuf.at[0], acc.at[pl.ds(base_r + m * K, K)])
            rem = last % K
            pltpu.sync_copy(
                gbuf.at[0, pl.ds(0, rem)],
                acc.at[pl.ds(base_r + (last // K) * K, rem)],
            )

        plsc.subcore_barrier()

        def run_group(p, nb):
            cps = [
                pltpu.async_copy(u_hbm.at[idxv.at[p, b, 0]], gbuf.at[b], gsem)
                for b in range(nb)
            ]
            scps = []
            for b in range(nb):
                cps[b].wait()
                scps.append(
                    pltpu.async_copy(gbuf.at[b], acc.at[idxv.at[p, b, 1]], ssem, add=True)
                )
            for s in scps:
                s.wait()

        def pair(h, carry):
            for p in range(2):
                g = 2 * h + p
                wait_idx(p)
                run_group(p, NBUF)
                pltpu.async_copy(
                    idx_hbm.at[wid, pl.ds((g + 2) * NBUF, NBUF)], idxv.at[p], isem
                )
            return carry

        lax.fori_loop(0, npair, pair, 0)
        wait_idx(0)
        run_group(0, NBUF)
        wait_idx(1)
        run_group(1, 1)
        plsc.subcore_barrier()

        @pl.when(sid < NS - 1)
        def _():
            pltpu.sync_copy(
                acc.at[pl.ds(sid * rpt, rpt)], out_hbm.at[cid, pl.ds(sid * rpt, rpt)]
            )

        @pl.when(sid == NS - 1)
        def _():
            pltpu.sync_copy(
                acc.at[pl.ds((NS - 1) * rpt, last)],
                out_hbm.at[cid, pl.ds((NS - 1) * rpt, last)],
            )

    return k(u, idx3)


def _tc_matmul(x, W):
    n, c_dim = x.shape

    def k(x_ref, w_ref, h_ref):
        h_ref[...] = jnp.dot(x_ref[...], w_ref[...], preferred_element_type=jnp.float32)

    return pl.pallas_call(
        k, out_shape=jax.ShapeDtypeStruct((n, W.shape[1]), jnp.float32)
    )(x, W)


def _tc_prep(deg_part, h):
    n, c_dim = h.shape

    nw = deg_part.shape[0]

    def k(dp_ref, h_ref, u_ref, dinv_ref):
        ones = jnp.ones((nw, 1), jnp.float32)
        deg = 1.0 + lax.dot_general(
            dp_ref[...], ones, (((0,), (0,)), ((), ())),
            preferred_element_type=jnp.float32,
        )
        dinv = lax.rsqrt(deg)
        u_ref[...] = h_ref[...] * dinv
        dinv_ref[...] = dinv

    return pl.pallas_call(
        k,
        out_shape=[
            jax.ShapeDtypeStruct((n, c_dim), jnp.float32),
            jax.ShapeDtypeStruct((n, 1), jnp.float32),
        ],
    )(deg_part, h)


def _tc_finale(agg, h, dinv, b, gamma, beta):
    n, c_dim = h.shape
    blk = 1000
    nblk = n // blk
    inv_n = 1.0 / n

    def k(agg_ref, h_ref, dinv_ref, b_ref, g_ref, bt_ref, o_ref, ybuf, st_ref):
        p = pl.program_id(0)
        i = pl.program_id(1)

        @pl.when(p == 0)
        def _():
            dv = dinv_ref[...]
            y = (agg_ref[0] + agg_ref[1]) * dv + h_ref[...] * (dv * dv) + b_ref[...]
            ybuf[pl.ds(i * blk, blk), :] = y

            @pl.when(i == 0)
            def _():
                st_ref[...] = jnp.zeros_like(st_ref)

            st_ref[0:1, :] += jnp.sum(y, axis=0, keepdims=True)
            st_ref[1:2, :] += jnp.sum(y * y, axis=0, keepdims=True)
            o_ref[...] = y

        @pl.when(p == 1)
        def _():
            mean = st_ref[0:1, :] * inv_n
            var = st_ref[1:2, :] * inv_n - mean * mean
            scale = lax.rsqrt(var + EPS) * g_ref[...]
            y = ybuf[pl.ds(i * blk, blk), :]
            o_ref[...] = jnp.maximum((y - mean) * scale + bt_ref[...], 0.0)

    return pl.pallas_call(
        k,
        grid=(2, nblk),
        in_specs=[
            pl.BlockSpec((2, blk, c_dim), lambda p, i: (0, i * (1 - p), 0)),
            pl.BlockSpec((blk, c_dim), lambda p, i: (i * (1 - p), 0)),
            pl.BlockSpec((blk, 1), lambda p, i: (i * (1 - p), 0)),
            pl.BlockSpec((1, c_dim), lambda p, i: (0, 0)),
            pl.BlockSpec((1, c_dim), lambda p, i: (0, 0)),
            pl.BlockSpec((1, c_dim), lambda p, i: (0, 0)),
        ],
        out_specs=pl.BlockSpec((blk, c_dim), lambda p, i: (i, 0)),
        out_shape=jax.ShapeDtypeStruct((n, c_dim), jnp.float32),
        scratch_shapes=[
            pltpu.VMEM((n, c_dim), jnp.float32),
            pltpu.VMEM((2, c_dim), jnp.float32),
        ],
    )(agg, h, dinv, b, gamma, beta)


def kernel(x, edge_index, W, b, gamma, beta):
    n = x.shape[0]
    row = edge_index[0]
    col = edge_index[1]
    h = _tc_matmul(x, W)
    deg_part = _sc_degree(col, n)
    u, dinv = _tc_prep(deg_part, h)
    agg = _sc_aggregate(u, row, col, n)
    return _tc_finale(
        agg, h, dinv, b.reshape(1, -1), gamma.reshape(1, -1), beta.reshape(1, -1)
    )


# speedup vs baseline: 1.0249x; 1.0249x over previous
import functools

import jax
import jax.numpy as jnp
from jax import lax
from jax.experimental import pallas as pl
from jax.experimental.pallas import tpu as pltpu
from jax.experimental.pallas import tpu_sc as plsc

NC = 2
NS = 16
L = 16
EPS = 1e-5


def _sc_degree(col, n):
    e = col.shape[0]
    nw = NC * NS
    ec = e // nw
    mesh = plsc.VectorSubcoreMesh(
        core_axis_name="c", subcore_axis_name="s", num_cores=NC, num_subcores=NS
    )

    @functools.partial(
        pl.kernel,
        mesh=mesh,
        out_type=jax.ShapeDtypeStruct((nw * n,), jnp.float32),
        scratch_types=[
            pltpu.VMEM((ec,), jnp.int32),
            pltpu.VMEM((n,), jnp.float32),
        ],
        compiler_params=pltpu.CompilerParams(needs_layout_passes=False),
    )
    def k(col_hbm, deg_hbm, colv, degv):
        wid = lax.axis_index("s") * NC + lax.axis_index("c")
        pltpu.sync_copy(col_hbm.at[pl.ds(wid * ec, ec)], colv)

        def zbody(i, carry):
            degv[pl.ds(i * L, L)] = jnp.zeros((L,), jnp.float32)
            return carry

        lax.fori_loop(0, n // L, zbody, 0, unroll=4)

        ones = jnp.ones((L,), jnp.float32)

        def cbody(i, carry):
            idx = colv[pl.ds(i * L, L)]
            plsc.addupdate_scatter(degv, [idx], ones)
            return carry

        lax.fori_loop(0, ec // L, cbody, 0, unroll=4)
        pltpu.sync_copy(degv, deg_hbm.at[pl.ds(wid * n, n)])

    return k(col).reshape(nw, n)


def _sc_aggregate(u, row, col, n):
    e = row.shape[0]
    c_dim = u.shape[1]
    ec = e // (NC * NS)
    K = 80
    nchunk = ec // K
    NBUF = 4
    ngrp = nchunk // NBUF
    assert nchunk % NBUF == 1 and (ngrp - 1) % 2 == 0
    npair = (ngrp - 1) // 2
    rpt = 632
    last = n - rpt * (NS - 1)
    mesh = plsc.VectorSubcoreMesh(
        core_axis_name="c", subcore_axis_name="s", num_cores=NC, num_subcores=NS
    )

    nci = nchunk + 3
    idx3 = jnp.zeros((NC * NS, nci, 2, K), jnp.int32)
    idx3 = idx3.at[:, :nchunk].set(
        jnp.stack(
            [row.reshape(NC * NS, nchunk, K), col.reshape(NC * NS, nchunk, K)],
            axis=2,
        )
    )

    @functools.partial(
        pl.kernel,
        mesh=mesh,
        out_type=jax.ShapeDtypeStruct((NC, n, c_dim), jnp.float32),
        scratch_types=[
            pltpu.VMEM((2, NBUF, 2, K), jnp.int32),
            pltpu.VMEM((NBUF, K, c_dim), jnp.float32),
            pltpu.VMEM_SHARED((n, c_dim), jnp.float32),
            pltpu.SemaphoreType.DMA,
            pltpu.SemaphoreType.DMA,
            pltpu.SemaphoreType.DMA,
        ],
        compiler_params=pltpu.CompilerParams(needs_layout_passes=False),
    )
    def k(u_hbm, idx_hbm, out_hbm, idxv, gbuf, acc, gsem, ssem, isem):
        cid = lax.axis_index("c")
        sid = lax.axis_index("s")
        wid = cid * NS + sid

        pltpu.async_copy(idx_hbm.at[wid, pl.ds(0, NBUF)], idxv.at[0], isem)
        pltpu.async_copy(idx_hbm.at[wid, pl.ds(NBUF, NBUF)], idxv.at[1], isem)

        def wait_idx(p):
            pltpu.make_async_copy(
                idx_hbm.at[wid, pl.ds(0, NBUF)], idxv.at[p], isem
            ).wait()

        nvec = c_dim // L

        def zbody(i, carry):
            r = i // nvec
            col0 = (i % nvec) * L
            gbuf[0, r, pl.ds(col0, L)] = jnp.zeros((L,), jnp.float32)
            return carry

        lax.fori_loop(0, K * nvec, zbody, 0, unroll=4)
        base_r = sid * rpt

        @pl.when(sid < NS - 1)
        def _():
            for m in range(rpt // K):
                pltpu.sync_copy(gbuf.at[0], acc.at[pl.ds(base_r + m * K, K)])
            rem = rpt % K
            pltpu.sync_copy(
                gbuf.at[0, pl.ds(0, rem)],
                acc.at[pl.ds(base_r + (rpt // K) * K, rem)],
            )

        @pl.when(sid == NS - 1)
        def _():
            for m in range(last // K):
                pltpu.sync_copy(gbuf.at[0], acc.at[pl.ds(base_r + m * K, K)])
            rem = last % K
            pltpu.sync_copy(
                gbuf.at[0, pl.ds(0, rem)],
                acc.at[pl.ds(base_r + (last // K) * K, rem)],
            )

        plsc.subcore_barrier()

        def run_group(p, nb):
            cps = [
                pltpu.async_copy(u_hbm.at[idxv.at[p, b, 0]], gbuf.at[b], gsem)
                for b in range(nb)
            ]
            scps = []
            for b in range(nb):
                cps[b].wait()
                scps.append(
                    pltpu.async_copy(gbuf.at[b], acc.at[idxv.at[p, b, 1]], ssem, add=True)
                )
            for s in scps:
                s.wait()

        def pair(h, carry):
            for p in range(2):
                g = 2 * h + p
                wait_idx(p)
                run_group(p, NBUF)
                pltpu.async_copy(
                    idx_hbm.at[wid, pl.ds((g + 2) * NBUF, NBUF)], idxv.at[p], isem
                )
            return carry

        lax.fori_loop(0, npair, pair, 0)
        wait_idx(0)
        run_group(0, NBUF)
        wait_idx(1)
        run_group(1, 1)
        plsc.subcore_barrier()

        @pl.when(sid < NS - 1)
        def _():
            pltpu.sync_copy(
                acc.at[pl.ds(sid * rpt, rpt)], out_hbm.at[cid, pl.ds(sid * rpt, rpt)]
            )

        @pl.when(sid == NS - 1)
        def _():
            pltpu.sync_copy(
                acc.at[pl.ds((NS - 1) * rpt, last)],
                out_hbm.at[cid, pl.ds((NS - 1) * rpt, last)],
            )

    return k(u, idx3)


def _tc_prep(deg_part, h):
    n, c_dim = h.shape

    nw = deg_part.shape[0]

    def k(dp_ref, h_ref, u_ref, dinv_ref):
        ones = jnp.ones((nw, 1), jnp.float32)
        deg = 1.0 + lax.dot_general(
            dp_ref[...], ones, (((0,), (0,)), ((), ())),
            preferred_element_type=jnp.float32,
        )
        dinv = lax.rsqrt(deg)
        u_ref[...] = h_ref[...] * dinv
        dinv_ref[...] = dinv

    return pl.pallas_call(
        k,
        out_shape=[
            jax.ShapeDtypeStruct((n, c_dim), jnp.float32),
            jax.ShapeDtypeStruct((n, 1), jnp.float32),
        ],
    )(deg_part, h)


def _tc_finale(agg, x, dinv, W, b, gamma, beta):
    n, c_dim = x.shape
    c_out = W.shape[1]
    blk = 1000
    nblk = n // blk
    inv_n = 1.0 / n

    def k(agg_ref, x_ref, dinv_ref, w_ref, b_ref, g_ref, bt_ref, o_ref, ybuf, st_ref):
        p = pl.program_id(0)
        i = pl.program_id(1)

        @pl.when(p == 0)
        def _():
            dv = dinv_ref[...]
            pre = (agg_ref[0] + agg_ref[1]) * dv + x_ref[...] * (dv * dv)
            y = (
                jnp.dot(pre, w_ref[...], preferred_element_type=jnp.float32)
                + b_ref[...]
            )
            ybuf[pl.ds(i * blk, blk), :] = y

            @pl.when(i == 0)
            def _():
                st_ref[...] = jnp.zeros_like(st_ref)

            st_ref[0:1, :] += jnp.sum(y, axis=0, keepdims=True)
            st_ref[1:2, :] += jnp.sum(y * y, axis=0, keepdims=True)
            o_ref[...] = y

        @pl.when(p == 1)
        def _():
            mean = st_ref[0:1, :] * inv_n
            var = st_ref[1:2, :] * inv_n - mean * mean
            scale = lax.rsqrt(var + EPS) * g_ref[...]
            y = ybuf[pl.ds(i * blk, blk), :]
            o_ref[...] = jnp.maximum((y - mean) * scale + bt_ref[...], 0.0)

    return pl.pallas_call(
        k,
        grid=(2, nblk),
        in_specs=[
            pl.BlockSpec((2, blk, c_dim), lambda p, i: (0, i * (1 - p), 0)),
            pl.BlockSpec((blk, c_dim), lambda p, i: (i * (1 - p), 0)),
            pl.BlockSpec((blk, 1), lambda p, i: (i * (1 - p), 0)),
            pl.BlockSpec((c_dim, c_out), lambda p, i: (0, 0)),
            pl.BlockSpec((1, c_out), lambda p, i: (0, 0)),
            pl.BlockSpec((1, c_out), lambda p, i: (0, 0)),
            pl.BlockSpec((1, c_out), lambda p, i: (0, 0)),
        ],
        out_specs=pl.BlockSpec((blk, c_out), lambda p, i: (i, 0)),
        out_shape=jax.ShapeDtypeStruct((n, c_out), jnp.float32),
        scratch_shapes=[
            pltpu.VMEM((n, c_out), jnp.float32),
            pltpu.VMEM((2, c_out), jnp.float32),
        ],
    )(agg, x, dinv, W, b, gamma, beta)


def kernel(x, edge_index, W, b, gamma, beta):
    n = x.shape[0]
    row = edge_index[0]
    col = edge_index[1]
    deg_part = _sc_degree(col, n)
    u, dinv = _tc_prep(deg_part, x)
    agg = _sc_aggregate(u, row, col, n)
    return _tc_finale(
        agg, x, dinv, W, b.reshape(1, -1), gamma.reshape(1, -1), beta.reshape(1, -1)
    )


# speedup vs baseline: 1.0345x; 1.0094x over previous
import functools

import jax
import jax.numpy as jnp
from jax import lax
from jax.experimental import pallas as pl
from jax.experimental.pallas import tpu as pltpu
from jax.experimental.pallas import tpu_sc as plsc

NC = 2
NS = 16
L = 16
EPS = 1e-5


def _sc_degree(col, n):
    e = col.shape[0]
    nw = NC * NS
    ec = e // nw
    mesh = plsc.VectorSubcoreMesh(
        core_axis_name="c", subcore_axis_name="s", num_cores=NC, num_subcores=NS
    )

    @functools.partial(
        pl.kernel,
        mesh=mesh,
        out_type=jax.ShapeDtypeStruct((nw * n,), jnp.float32),
        scratch_types=[
            pltpu.VMEM((ec,), jnp.int32),
            pltpu.VMEM((n,), jnp.float32),
        ],
        compiler_params=pltpu.CompilerParams(needs_layout_passes=False),
    )
    def k(col_hbm, deg_hbm, colv, degv):
        wid = lax.axis_index("s") * NC + lax.axis_index("c")
        pltpu.sync_copy(col_hbm.at[pl.ds(wid * ec, ec)], colv)

        def zbody(i, carry):
            degv[pl.ds(i * L, L)] = jnp.zeros((L,), jnp.float32)
            return carry

        lax.fori_loop(0, n // L, zbody, 0, unroll=4)

        ones = jnp.ones((L,), jnp.float32)

        def cbody(i, carry):
            idx = colv[pl.ds(i * L, L)]
            plsc.addupdate_scatter(degv, [idx], ones)
            return carry

        lax.fori_loop(0, ec // L, cbody, 0, unroll=4)
        pltpu.sync_copy(degv, deg_hbm.at[pl.ds(wid * n, n)])

    return k(col).reshape(nw, n)


def _sc_aggregate(u, row, col, n):
    e = row.shape[0]
    c_dim = u.shape[1]
    ec = e // (NC * NS)
    K = 80
    nchunk = ec // K
    NBUF = 4
    ngrp = nchunk // NBUF
    assert nchunk % NBUF == 1 and (ngrp - 1) % 2 == 0
    npair = (ngrp - 1) // 2
    rpt = 632
    last = n - rpt * (NS - 1)
    mesh = plsc.VectorSubcoreMesh(
        core_axis_name="c", subcore_axis_name="s", num_cores=NC, num_subcores=NS
    )

    nci = nchunk + 3
    idx3 = jnp.zeros((NC * NS, nci, 2, K), jnp.int32)
    idx3 = idx3.at[:, :nchunk].set(
        jnp.stack(
            [row.reshape(NC * NS, nchunk, K), col.reshape(NC * NS, nchunk, K)],
            axis=2,
        )
    )

    @functools.partial(
        pl.kernel,
        mesh=mesh,
        out_type=jax.ShapeDtypeStruct((NC, n, c_dim), jnp.float32),
        scratch_types=[
            pltpu.VMEM((2, NBUF, 2, K), jnp.int32),
            pltpu.VMEM((NBUF, K, c_dim), jnp.float32),
            pltpu.VMEM_SHARED((n, c_dim), jnp.float32),
            pltpu.SemaphoreType.DMA,
            pltpu.SemaphoreType.DMA,
            pltpu.SemaphoreType.DMA,
        ],
        compiler_params=pltpu.CompilerParams(needs_layout_passes=False),
    )
    def k(u_hbm, idx_hbm, out_hbm, idxv, gbuf, acc, gsem, ssem, isem):
        cid = lax.axis_index("c")
        sid = lax.axis_index("s")
        wid = cid * NS + sid

        pltpu.async_copy(idx_hbm.at[wid, pl.ds(0, NBUF)], idxv.at[0], isem)
        pltpu.async_copy(idx_hbm.at[wid, pl.ds(NBUF, NBUF)], idxv.at[1], isem)

        def wait_idx(p):
            pltpu.make_async_copy(
                idx_hbm.at[wid, pl.ds(0, NBUF)], idxv.at[p], isem
            ).wait()

        nvec = c_dim // L

        def zbody(i, carry):
            r = i // nvec
            col0 = (i % nvec) * L
            gbuf[0, r, pl.ds(col0, L)] = jnp.zeros((L,), jnp.float32)
            return carry

        lax.fori_loop(0, K * nvec, zbody, 0, unroll=4)
        base_r = sid * rpt

        @pl.when(sid < NS - 1)
        def _():
            zcps = [
                pltpu.async_copy(
                    gbuf.at[0], acc.at[pl.ds(base_r + m * K, K)], ssem
                )
                for m in range(rpt // K)
            ]
            rem = rpt % K
            zcps.append(
                pltpu.async_copy(
                    gbuf.at[0, pl.ds(0, rem)],
                    acc.at[pl.ds(base_r + (rpt // K) * K, rem)],
                    ssem,
                )
            )
            for z in zcps:
                z.wait()

        @pl.when(sid == NS - 1)
        def _():
            zcps = [
                pltpu.async_copy(
                    gbuf.at[0], acc.at[pl.ds(base_r + m * K, K)], ssem
                )
                for m in range(last // K)
            ]
            rem = last % K
            zcps.append(
                pltpu.async_copy(
                    gbuf.at[0, pl.ds(0, rem)],
                    acc.at[pl.ds(base_r + (last // K) * K, rem)],
                    ssem,
                )
            )
            for z in zcps:
                z.wait()

        plsc.subcore_barrier()

        def run_group(p, nb):
            cps = [
                pltpu.async_copy(u_hbm.at[idxv.at[p, b, 0]], gbuf.at[b], gsem)
                for b in range(nb)
            ]
            scps = []
            for b in range(nb):
                cps[b].wait()
                scps.append(
                    pltpu.async_copy(gbuf.at[b], acc.at[idxv.at[p, b, 1]], ssem, add=True)
                )
            for s in scps:
                s.wait()

        def pair(h, carry):
            for p in range(2):
                g = 2 * h + p
                wait_idx(p)
                run_group(p, NBUF)
                pltpu.async_copy(
                    idx_hbm.at[wid, pl.ds((g + 2) * NBUF, NBUF)], idxv.at[p], isem
                )
            return carry

        lax.fori_loop(0, npair, pair, 0)
        wait_idx(0)
        run_group(0, NBUF)
        wait_idx(1)
        run_group(1, 1)
        plsc.subcore_barrier()

        @pl.when(sid < NS - 1)
        def _():
            pltpu.sync_copy(
                acc.at[pl.ds(sid * rpt, rpt)], out_hbm.at[cid, pl.ds(sid * rpt, rpt)]
            )

        @pl.when(sid == NS - 1)
        def _():
            pltpu.sync_copy(
                acc.at[pl.ds((NS - 1) * rpt, last)],
                out_hbm.at[cid, pl.ds((NS - 1) * rpt, last)],
            )

    return k(u, idx3)


def _tc_prep(deg_part, h):
    n, c_dim = h.shape

    nw = deg_part.shape[0]

    def k(dp_ref, h_ref, u_ref, dinv_ref):
        ones = jnp.ones((nw, 1), jnp.float32)
        deg = 1.0 + lax.dot_general(
            dp_ref[...], ones, (((0,), (0,)), ((), ())),
            preferred_element_type=jnp.float32,
        )
        dinv = lax.rsqrt(deg)
        u_ref[...] = h_ref[...] * dinv
        dinv_ref[...] = dinv

    return pl.pallas_call(
        k,
        out_shape=[
            jax.ShapeDtypeStruct((n, c_dim), jnp.float32),
            jax.ShapeDtypeStruct((n, 1), jnp.float32),
        ],
    )(deg_part, h)


def _tc_finale(agg, x, dinv, W, b, gamma, beta):
    n, c_dim = x.shape
    c_out = W.shape[1]
    blk = 1000
    nblk = n // blk
    inv_n = 1.0 / n

    def k(agg_ref, x_ref, dinv_ref, w_ref, b_ref, g_ref, bt_ref, o_ref, ybuf, st_ref):
        p = pl.program_id(0)
        i = pl.program_id(1)

        @pl.when(p == 0)
        def _():
            dv = dinv_ref[...]
            pre = (agg_ref[0] + agg_ref[1]) * dv + x_ref[...] * (dv * dv)
            y = (
                jnp.dot(pre, w_ref[...], preferred_element_type=jnp.float32)
                + b_ref[...]
            )
            ybuf[pl.ds(i * blk, blk), :] = y

            @pl.when(i == 0)
            def _():
                st_ref[...] = jnp.zeros_like(st_ref)

            st_ref[0:1, :] += jnp.sum(y, axis=0, keepdims=True)
            st_ref[1:2, :] += jnp.sum(y * y, axis=0, keepdims=True)

        @pl.when(p == 1)
        def _():
            mean = st_ref[0:1, :] * inv_n
            var = st_ref[1:2, :] * inv_n - mean * mean
            scale = lax.rsqrt(var + EPS) * g_ref[...]
            y = ybuf[pl.ds(i * blk, blk), :]
            o_ref[...] = jnp.maximum((y - mean) * scale + bt_ref[...], 0.0)

    return pl.pallas_call(
        k,
        grid=(2, nblk),
        in_specs=[
            pl.BlockSpec((2, blk, c_dim), lambda p, i: (0, i * (1 - p), 0)),
            pl.BlockSpec((blk, c_dim), lambda p, i: (i * (1 - p), 0)),
            pl.BlockSpec((blk, 1), lambda p, i: (i * (1 - p), 0)),
            pl.BlockSpec((c_dim, c_out), lambda p, i: (0, 0)),
            pl.BlockSpec((1, c_out), lambda p, i: (0, 0)),
            pl.BlockSpec((1, c_out), lambda p, i: (0, 0)),
            pl.BlockSpec((1, c_out), lambda p, i: (0, 0)),
        ],
        out_specs=pl.BlockSpec((blk, c_out), lambda p, i: (i * p, 0)),
        out_shape=jax.ShapeDtypeStruct((n, c_out), jnp.float32),
        scratch_shapes=[
            pltpu.VMEM((n, c_out), jnp.float32),
            pltpu.VMEM((2, c_out), jnp.float32),
        ],
    )(agg, x, dinv, W, b, gamma, beta)


def kernel(x, edge_index, W, b, gamma, beta):
    n = x.shape[0]
    row = edge_index[0]
    col = edge_index[1]
    deg_part = _sc_degree(col, n)
    u, dinv = _tc_prep(deg_part, x)
    agg = _sc_aggregate(u, row, col, n)
    return _tc_finale(
        agg, x, dinv, W, b.reshape(1, -1), gamma.reshape(1, -1), beta.reshape(1, -1)
    )


# speedup vs baseline: 1.0574x; 1.0222x over previous
import functools

import jax
import jax.numpy as jnp
from jax import lax
from jax.experimental import pallas as pl
from jax.experimental.pallas import tpu as pltpu
from jax.experimental.pallas import tpu_sc as plsc

NC = 2
NS = 16
L = 16
EPS = 1e-5


def _sc_degree(col, n):
    e = col.shape[0]
    nw = NC * NS
    ec = e // nw
    mesh = plsc.VectorSubcoreMesh(
        core_axis_name="c", subcore_axis_name="s", num_cores=NC, num_subcores=NS
    )

    @functools.partial(
        pl.kernel,
        mesh=mesh,
        out_type=jax.ShapeDtypeStruct((nw * n,), jnp.float32),
        scratch_types=[
            pltpu.VMEM((ec,), jnp.int32),
            pltpu.VMEM((n,), jnp.float32),
        ],
        compiler_params=pltpu.CompilerParams(needs_layout_passes=False),
    )
    def k(col_hbm, deg_hbm, colv, degv):
        wid = lax.axis_index("s") * NC + lax.axis_index("c")
        pltpu.sync_copy(col_hbm.at[pl.ds(wid * ec, ec)], colv)

        def zbody(i, carry):
            degv[pl.ds(i * L, L)] = jnp.zeros((L,), jnp.float32)
            return carry

        lax.fori_loop(0, n // L, zbody, 0, unroll=4)

        ones = jnp.ones((L,), jnp.float32)

        def cbody(i, carry):
            idx = colv[pl.ds(i * L, L)]
            plsc.addupdate_scatter(degv, [idx], ones)
            return carry

        lax.fori_loop(0, ec // L, cbody, 0, unroll=4)
        pltpu.sync_copy(degv, deg_hbm.at[pl.ds(wid * n, n)])

    return k(col).reshape(nw, n)


def _sc_aggregate(u, row, col, n):
    e = row.shape[0]
    c_dim = u.shape[1]
    ec = e // (NC * NS)
    K = 80
    nchunk = ec // K
    NBUF = 4
    ngrp = nchunk // NBUF
    assert nchunk % NBUF == 1 and (ngrp - 1) % 2 == 0
    npair = (ngrp - 1) // 2
    rpt = 632
    last = n - rpt * (NS - 1)
    mesh = plsc.VectorSubcoreMesh(
        core_axis_name="c", subcore_axis_name="s", num_cores=NC, num_subcores=NS
    )

    nci = nchunk + 3
    idx3 = jnp.zeros((NC * NS, nci, 2, K), jnp.int32)
    idx3 = idx3.at[:, :nchunk].set(
        jnp.stack(
            [row.reshape(NC * NS, nchunk, K), col.reshape(NC * NS, nchunk, K)],
            axis=2,
        )
    )

    @functools.partial(
        pl.kernel,
        mesh=mesh,
        out_type=jax.ShapeDtypeStruct((NC, n, c_dim), jnp.float32),
        scratch_types=[
            pltpu.VMEM((2, NBUF, 2, K), jnp.int32),
            pltpu.VMEM((NBUF, K, c_dim), jnp.float32),
            pltpu.VMEM_SHARED((n, c_dim), jnp.float32),
            pltpu.SemaphoreType.DMA,
            pltpu.SemaphoreType.DMA,
            pltpu.SemaphoreType.DMA,
        ],
        compiler_params=pltpu.CompilerParams(needs_layout_passes=False),
    )
    def k(u_hbm, idx_hbm, out_hbm, idxv, gbuf, acc, gsem, ssem, isem):
        cid = lax.axis_index("c")
        sid = lax.axis_index("s")
        wid = cid * NS + sid

        pltpu.async_copy(idx_hbm.at[wid, pl.ds(0, NBUF)], idxv.at[0], isem)
        pltpu.async_copy(idx_hbm.at[wid, pl.ds(NBUF, NBUF)], idxv.at[1], isem)

        def wait_idx(p):
            pltpu.make_async_copy(
                idx_hbm.at[wid, pl.ds(0, NBUF)], idxv.at[p], isem
            ).wait()

        nvec = c_dim // L

        def zbody(i, carry):
            r = i // nvec
            col0 = (i % nvec) * L
            gbuf[0, r, pl.ds(col0, L)] = jnp.zeros((L,), jnp.float32)
            return carry

        lax.fori_loop(0, K * nvec, zbody, 0, unroll=4)
        base_r = sid * rpt

        @pl.when(sid < NS - 1)
        def _():
            zcps = [
                pltpu.async_copy(
                    gbuf.at[0], acc.at[pl.ds(base_r + m * K, K)], ssem
                )
                for m in range(rpt // K)
            ]
            rem = rpt % K
            zcps.append(
                pltpu.async_copy(
                    gbuf.at[0, pl.ds(0, rem)],
                    acc.at[pl.ds(base_r + (rpt // K) * K, rem)],
                    ssem,
                )
            )
            for z in zcps:
                z.wait()

        @pl.when(sid == NS - 1)
        def _():
            zcps = [
                pltpu.async_copy(
                    gbuf.at[0], acc.at[pl.ds(base_r + m * K, K)], ssem
                )
                for m in range(last // K)
            ]
            rem = last % K
            zcps.append(
                pltpu.async_copy(
                    gbuf.at[0, pl.ds(0, rem)],
                    acc.at[pl.ds(base_r + (last // K) * K, rem)],
                    ssem,
                )
            )
            for z in zcps:
                z.wait()

        plsc.subcore_barrier()

        def run_group(p, nb):
            cps = [
                pltpu.async_copy(u_hbm.at[idxv.at[p, b, 0]], gbuf.at[b], gsem)
                for b in range(nb)
            ]
            scps = []
            for b in range(nb):
                cps[b].wait()
                scps.append(
                    pltpu.async_copy(gbuf.at[b], acc.at[idxv.at[p, b, 1]], ssem, add=True)
                )
            for s in scps:
                s.wait()

        def pair(h, carry):
            for p in range(2):
                g = 2 * h + p
                wait_idx(p)
                run_group(p, NBUF)
                pltpu.async_copy(
                    idx_hbm.at[wid, pl.ds((g + 2) * NBUF, NBUF)], idxv.at[p], isem
                )
            return carry

        lax.fori_loop(0, npair, pair, 0)
        wait_idx(0)
        run_group(0, NBUF)
        wait_idx(1)
        run_group(1, 1)
        plsc.subcore_barrier()

        @pl.when(sid < NS - 1)
        def _():
            pltpu.sync_copy(
                acc.at[pl.ds(sid * rpt, rpt)], out_hbm.at[cid, pl.ds(sid * rpt, rpt)]
            )

        @pl.when(sid == NS - 1)
        def _():
            pltpu.sync_copy(
                acc.at[pl.ds((NS - 1) * rpt, last)],
                out_hbm.at[cid, pl.ds((NS - 1) * rpt, last)],
            )

    return k(u, idx3)


def _tc_prep(deg_part, h):
    n, c_dim = h.shape

    nw = deg_part.shape[0]

    def k(dp_ref, h_ref, u_ref, dinv_ref):
        ones = jnp.ones((nw, 1), jnp.float32)
        deg = 1.0 + lax.dot_general(
            dp_ref[...], ones, (((0,), (0,)), ((), ())),
            preferred_element_type=jnp.float32,
        )
        dinv = lax.rsqrt(deg)
        u_ref[...] = h_ref[...] * dinv
        dinv_ref[...] = dinv

    return pl.pallas_call(
        k,
        out_shape=[
            jax.ShapeDtypeStruct((n, c_dim), jnp.float32),
            jax.ShapeDtypeStruct((n, 1), jnp.float32),
        ],
    )(deg_part, h)


def _tc_finale(agg, x, dinv, W, b, gamma, beta):
    n, c_dim = x.shape
    c_out = W.shape[1]
    blk = 2000
    nblk = n // blk
    inv_n = 1.0 / n

    def k(agg_ref, x_ref, dinv_ref, w_ref, b_ref, g_ref, bt_ref, o_ref, ybuf, st_ref):
        p = pl.program_id(0)
        i = pl.program_id(1)

        @pl.when(p == 0)
        def _():
            dv = dinv_ref[...]
            pre = (agg_ref[0] + agg_ref[1]) * dv + x_ref[...] * (dv * dv)
            y = (
                jnp.dot(pre, w_ref[...], preferred_element_type=jnp.float32)
                + b_ref[...]
            )
            ybuf[pl.ds(i * blk, blk), :] = y

            @pl.when(i == 0)
            def _():
                st_ref[...] = jnp.zeros_like(st_ref)

            st_ref[0:1, :] += jnp.sum(y, axis=0, keepdims=True)
            st_ref[1:2, :] += jnp.sum(y * y, axis=0, keepdims=True)

        @pl.when(p == 1)
        def _():
            mean = st_ref[0:1, :] * inv_n
            var = st_ref[1:2, :] * inv_n - mean * mean
            scale = lax.rsqrt(var + EPS) * g_ref[...]
            y = ybuf[pl.ds(i * blk, blk), :]
            o_ref[...] = jnp.maximum((y - mean) * scale + bt_ref[...], 0.0)

    return pl.pallas_call(
        k,
        grid=(2, nblk),
        in_specs=[
            pl.BlockSpec((2, blk, c_dim), lambda p, i: (0, i * (1 - p), 0)),
            pl.BlockSpec((blk, c_dim), lambda p, i: (i * (1 - p), 0)),
            pl.BlockSpec((blk, 1), lambda p, i: (i * (1 - p), 0)),
            pl.BlockSpec((c_dim, c_out), lambda p, i: (0, 0)),
            pl.BlockSpec((1, c_out), lambda p, i: (0, 0)),
            pl.BlockSpec((1, c_out), lambda p, i: (0, 0)),
            pl.BlockSpec((1, c_out), lambda p, i: (0, 0)),
        ],
        out_specs=pl.BlockSpec((blk, c_out), lambda p, i: (i * p, 0)),
        out_shape=jax.ShapeDtypeStruct((n, c_out), jnp.float32),
        scratch_shapes=[
            pltpu.VMEM((n, c_out), jnp.float32),
            pltpu.VMEM((2, c_out), jnp.float32),
        ],
    )(agg, x, dinv, W, b, gamma, beta)


def kernel(x, edge_index, W, b, gamma, beta):
    n = x.shape[0]
    row = edge_index[0]
    col = edge_index[1]
    deg_part = _sc_degree(col, n)
    u, dinv = _tc_prep(deg_part, x)
    agg = _sc_aggregate(u, row, col, n)
    return _tc_finale(
        agg, x, dinv, W, b.reshape(1, -1), gamma.reshape(1, -1), beta.reshape(1, -1)
    )


# speedup vs baseline: 1.0587x; 1.0012x over previous
import functools

import jax
import jax.numpy as jnp
from jax import lax
from jax.experimental import pallas as pl
from jax.experimental.pallas import tpu as pltpu
from jax.experimental.pallas import tpu_sc as plsc

NC = 2
NS = 16
L = 16
EPS = 1e-5


def _sc_degree(col, n):
    e = col.shape[0]
    nw = NC * NS
    ec = e // nw
    mesh = plsc.VectorSubcoreMesh(
        core_axis_name="c", subcore_axis_name="s", num_cores=NC, num_subcores=NS
    )

    @functools.partial(
        pl.kernel,
        mesh=mesh,
        out_type=jax.ShapeDtypeStruct((nw * n,), jnp.float32),
        scratch_types=[
            pltpu.VMEM((ec,), jnp.int32),
            pltpu.VMEM((n,), jnp.float32),
        ],
        compiler_params=pltpu.CompilerParams(needs_layout_passes=False),
    )
    def k(col_hbm, deg_hbm, colv, degv):
        wid = lax.axis_index("s") * NC + lax.axis_index("c")
        pltpu.sync_copy(col_hbm.at[pl.ds(wid * ec, ec)], colv)

        def zbody(i, carry):
            degv[pl.ds(i * L, L)] = jnp.zeros((L,), jnp.float32)
            return carry

        lax.fori_loop(0, n // L, zbody, 0, unroll=4)

        ones = jnp.ones((L,), jnp.float32)

        def cbody(i, carry):
            idx = colv[pl.ds(i * L, L)]
            plsc.addupdate_scatter(degv, [idx], ones)
            return carry

        lax.fori_loop(0, ec // L, cbody, 0, unroll=4)
        pltpu.sync_copy(degv, deg_hbm.at[pl.ds(wid * n, n)])

    return k(col).reshape(nw, n)


def _sc_aggregate(u, row, col, n):
    e = row.shape[0]
    c_dim = u.shape[1]
    ec = e // (NC * NS)
    K = 80
    nchunk = ec // K
    NBUF = 4
    ngrp = nchunk // NBUF
    assert nchunk % NBUF == 1 and (ngrp - 1) % 2 == 0
    npair = (ngrp - 1) // 2
    rpt = 632
    last = n - rpt * (NS - 1)
    mesh = plsc.VectorSubcoreMesh(
        core_axis_name="c", subcore_axis_name="s", num_cores=NC, num_subcores=NS
    )

    nci = nchunk + 3
    idx3 = jnp.zeros((NC * NS, nci, 2, K), jnp.int32)
    idx3 = idx3.at[:, :nchunk].set(
        jnp.stack(
            [row.reshape(NC * NS, nchunk, K), col.reshape(NC * NS, nchunk, K)],
            axis=2,
        )
    )

    @functools.partial(
        pl.kernel,
        mesh=mesh,
        out_type=jax.ShapeDtypeStruct((NC, n, c_dim), jnp.float32),
        scratch_types=[
            pltpu.VMEM((2, NBUF, 2, K), jnp.int32),
            pltpu.VMEM((NBUF, K, c_dim), jnp.float32),
            pltpu.VMEM_SHARED((n, c_dim), jnp.float32),
            pltpu.SemaphoreType.DMA,
            pltpu.SemaphoreType.DMA,
            pltpu.SemaphoreType.DMA,
        ],
        compiler_params=pltpu.CompilerParams(needs_layout_passes=False),
    )
    def k(u_hbm, idx_hbm, out_hbm, idxv, gbuf, acc, gsem, ssem, isem):
        cid = lax.axis_index("c")
        sid = lax.axis_index("s")
        wid = cid * NS + sid

        pltpu.async_copy(idx_hbm.at[wid, pl.ds(0, NBUF)], idxv.at[0], isem)
        pltpu.async_copy(idx_hbm.at[wid, pl.ds(NBUF, NBUF)], idxv.at[1], isem)

        def wait_idx(p):
            pltpu.make_async_copy(
                idx_hbm.at[wid, pl.ds(0, NBUF)], idxv.at[p], isem
            ).wait()

        nvec = c_dim // L

        def zbody(i, carry):
            r = i // nvec
            col0 = (i % nvec) * L
            gbuf[0, r, pl.ds(col0, L)] = jnp.zeros((L,), jnp.float32)
            return carry

        lax.fori_loop(0, K * nvec, zbody, 0, unroll=4)
        base_r = sid * rpt

        @pl.when(sid < NS - 1)
        def _():
            zcps = [
                pltpu.async_copy(
                    gbuf.at[0], acc.at[pl.ds(base_r + m * K, K)], ssem
                )
                for m in range(rpt // K)
            ]
            rem = rpt % K
            zcps.append(
                pltpu.async_copy(
                    gbuf.at[0, pl.ds(0, rem)],
                    acc.at[pl.ds(base_r + (rpt // K) * K, rem)],
                    ssem,
                )
            )
            for z in zcps:
                z.wait()

        @pl.when(sid == NS - 1)
        def _():
            zcps = [
                pltpu.async_copy(
                    gbuf.at[0], acc.at[pl.ds(base_r + m * K, K)], ssem
                )
                for m in range(last // K)
            ]
            rem = last % K
            zcps.append(
                pltpu.async_copy(
                    gbuf.at[0, pl.ds(0, rem)],
                    acc.at[pl.ds(base_r + (last // K) * K, rem)],
                    ssem,
                )
            )
            for z in zcps:
                z.wait()

        plsc.subcore_barrier()

        def run_group(p, nb):
            cps = [
                pltpu.async_copy(u_hbm.at[idxv.at[p, b, 0]], gbuf.at[b], gsem)
                for b in range(nb)
            ]
            scps = []
            for b in range(nb):
                cps[b].wait()
                scps.append(
                    pltpu.async_copy(gbuf.at[b], acc.at[idxv.at[p, b, 1]], ssem, add=True)
                )
            for s in scps:
                s.wait()

        def pair(h, carry):
            for p in range(2):
                g = 2 * h + p
                wait_idx(p)
                run_group(p, NBUF)
                pltpu.async_copy(
                    idx_hbm.at[wid, pl.ds((g + 2) * NBUF, NBUF)], idxv.at[p], isem
                )
            return carry

        lax.fori_loop(0, npair, pair, 0)
        wait_idx(0)
        run_group(0, NBUF)
        wait_idx(1)
        run_group(1, 1)
        plsc.subcore_barrier()

        @pl.when(sid < NS - 1)
        def _():
            pltpu.sync_copy(
                acc.at[pl.ds(sid * rpt, rpt)], out_hbm.at[cid, pl.ds(sid * rpt, rpt)]
            )

        @pl.when(sid == NS - 1)
        def _():
            pltpu.sync_copy(
                acc.at[pl.ds((NS - 1) * rpt, last)],
                out_hbm.at[cid, pl.ds((NS - 1) * rpt, last)],
            )

    return k(u, idx3)


def _tc_prep(deg_part, h):
    n, c_dim = h.shape

    nw = deg_part.shape[0]

    def k(dp_ref, h_ref, u_ref, dinv_ref):
        ones = jnp.ones((nw, 1), jnp.float32)
        deg = 1.0 + lax.dot_general(
            dp_ref[...], ones, (((0,), (0,)), ((), ())),
            preferred_element_type=jnp.float32,
        )
        dinv = lax.rsqrt(deg)
        u_ref[...] = h_ref[...] * dinv
        dinv_ref[...] = dinv

    return pl.pallas_call(
        k,
        out_shape=[
            jax.ShapeDtypeStruct((n, c_dim), jnp.float32),
            jax.ShapeDtypeStruct((n, 1), jnp.float32),
        ],
    )(deg_part, h)


def _tc_finale(agg, x, dinv, W, b, gamma, beta):
    n, c_dim = x.shape
    c_out = W.shape[1]
    blk = 5000
    nblk = n // blk
    inv_n = 1.0 / n

    def k(agg_ref, x_ref, dinv_ref, w_ref, b_ref, g_ref, bt_ref, o_ref, ybuf, st_ref):
        p = pl.program_id(0)
        i = pl.program_id(1)

        @pl.when(p == 0)
        def _():
            dv = dinv_ref[...]
            pre = (agg_ref[0] + agg_ref[1]) * dv + x_ref[...] * (dv * dv)
            y = (
                jnp.dot(pre, w_ref[...], preferred_element_type=jnp.float32)
                + b_ref[...]
            )
            ybuf[pl.ds(i * blk, blk), :] = y

            @pl.when(i == 0)
            def _():
                st_ref[...] = jnp.zeros_like(st_ref)

            st_ref[0:1, :] += jnp.sum(y, axis=0, keepdims=True)
            st_ref[1:2, :] += jnp.sum(y * y, axis=0, keepdims=True)

        @pl.when(p == 1)
        def _():
            mean = st_ref[0:1, :] * inv_n
            var = st_ref[1:2, :] * inv_n - mean * mean
            scale = lax.rsqrt(var + EPS) * g_ref[...]
            y = ybuf[pl.ds(i * blk, blk), :]
            o_ref[...] = jnp.maximum((y - mean) * scale + bt_ref[...], 0.0)

    return pl.pallas_call(
        k,
        grid=(2, nblk),
        in_specs=[
            pl.BlockSpec((2, blk, c_dim), lambda p, i: (0, i * (1 - p), 0)),
            pl.BlockSpec((blk, c_dim), lambda p, i: (i * (1 - p), 0)),
            pl.BlockSpec((blk, 1), lambda p, i: (i * (1 - p), 0)),
            pl.BlockSpec((c_dim, c_out), lambda p, i: (0, 0)),
            pl.BlockSpec((1, c_out), lambda p, i: (0, 0)),
            pl.BlockSpec((1, c_out), lambda p, i: (0, 0)),
            pl.BlockSpec((1, c_out), lambda p, i: (0, 0)),
        ],
        out_specs=pl.BlockSpec((blk, c_out), lambda p, i: (i * p, 0)),
        out_shape=jax.ShapeDtypeStruct((n, c_out), jnp.float32),
        scratch_shapes=[
            pltpu.VMEM((n, c_out), jnp.float32),
            pltpu.VMEM((2, c_out), jnp.float32),
        ],
    )(agg, x, dinv, W, b, gamma, beta)


def kernel(x, edge_index, W, b, gamma, beta):
    n = x.shape[0]
    row = edge_index[0]
    col = edge_index[1]
    deg_part = _sc_degree(col, n)
    u, dinv = _tc_prep(deg_part, x)
    agg = _sc_aggregate(u, row, col, n)
    return _tc_finale(
        agg, x, dinv, W, b.reshape(1, -1), gamma.reshape(1, -1), beta.reshape(1, -1)
    )
